# baseline (speedup 1.0000x reference)
import jax
import jax.numpy as jnp
from jax.experimental import pallas as pl
from jax.experimental.pallas import tpu as pltpu


def _pick_row_block(n, pref):
    for cand in pref:
        if n % cand == 0 and (n // cand) % 2 == 1:
            return cand
    return n


def _dinv(deg):
    return jnp.where(deg > 0, jax.lax.rsqrt(deg), 0.0)


def _make_kernel(lb, nlb, align):
    def _kernel(adj_ref, x_ref, w1_ref, b1_ref, w2_ref, b2_ref,
                out_ref, adjb_ref,
                y1_scr, y2_scr, stg, sem_out, sem_in):
        p = pl.program_id(0)
        i = pl.program_id(1)

        def rows(blk):
            return pl.ds(pl.multiple_of(blk * lb, align), lb)

        def out_copy(blk, buf):
            return pltpu.make_async_copy(
                stg.at[pl.ds(buf * lb, lb), :], adjb_ref.at[rows(blk), :],
                sem_out.at[buf])

        def in_copy(blk, buf):
            return pltpu.make_async_copy(
                adjb_ref.at[rows(blk), :], stg.at[pl.ds(buf * lb, lb), :],
                sem_in.at[buf])

        def p0_body(b):
            @pl.when(i >= 2)
            def _():
                out_copy(i - 2, b).wait()

            stg[pl.ds(b * lb, lb), :] = adj_ref[...].astype(jnp.bfloat16)
            out_copy(i, b).start()
            deg = jnp.sum(stg[pl.ds(b * lb, lb), :], axis=1,
                          dtype=jnp.float32).reshape(lb, 1) + 1.0
            dv = _dinv(deg)
            z = jnp.dot(x_ref[...], w1_ref[...],
                        preferred_element_type=jnp.float32)
            y1_scr[rows(i), :] = (dv * z).astype(jnp.bfloat16)

        @pl.when((p == 0) & (i % 2 == 0))
        def _():
            p0_body(0)

        @pl.when((p == 0) & (i % 2 == 1))
        def _():
            p0_body(1)

        @pl.when((p == 0) & (i == nlb - 1))
        def _():
            out_copy(nlb - 2, (nlb - 2) % 2).wait()
            in_copy(0, 1 - (nlb - 1) % 2).start()

        @pl.when(p > 0)
        def _():
            s = (p - 1) * nlb + i

            @pl.when((p == 1) & (i == 0))
            def _():
                out_copy(nlb - 1, (nlb - 1) % 2).wait()

            def stage_body(cbuf):
                @pl.when(s + 1 < 2 * nlb)
                def _():
                    in_copy((s + 1) % nlb, 1 - cbuf).start()

                in_copy(i, cbuf).wait()
                ab = stg[pl.ds(cbuf * lb, lb), :]
                deg = jnp.sum(stg[pl.ds(cbuf * lb, lb), :], axis=1,
                              dtype=jnp.float32).reshape(lb, 1) + 1.0
                dinv = _dinv(deg)

                @pl.when(p == 1)
                def _():
                    acc = jnp.dot(ab, y1_scr[...],
                                  preferred_element_type=jnp.float32)
                    yself = y1_scr[rows(i), :].astype(jnp.float32)
                    x1 = dinv * (acc + yself) + b1_ref[...]
                    x1 = jnp.maximum(x1, 0.0)
                    y2_scr[rows(i), :] = (
                        dinv * jnp.dot(x1, w2_ref[...],
                                       preferred_element_type=jnp.float32)
                    ).astype(jnp.bfloat16)

                @pl.when(p == 2)
                def _():
                    acc = jnp.dot(ab, y2_scr[...],
                                  preferred_element_type=jnp.float32)
                    yself = y2_scr[rows(i), :].astype(jnp.float32)
                    out_ref[...] = dinv * (acc + yself) + b2_ref[...]

            @pl.when((s + 1) % 2 == 0)
            def _():
                stage_body(0)

            @pl.when((s + 1) % 2 == 1)
            def _():
                stage_body(1)

    return _kernel


@jax.jit
def kernel(feature, adj, W1, b1, W2, b2):
    n, d = feature.shape
    h1 = W1.shape[1]
    h2 = W2.shape[1]
    lb = _pick_row_block(n, (400, 200, 80, 40, 16, 8))
    nlb = n // lb
    align = lb & (-lb)

    b1r = b1.reshape(1, h1)
    b2r = b2.reshape(1, h2)

    last = nlb - 1
    x2, _ = pl.pallas_call(
        _make_kernel(lb, nlb, align),
        grid=(3, nlb),
        in_specs=[
            pl.BlockSpec((lb, n), lambda p, i: (jnp.where(p == 0, i, last), 0)),
            pl.BlockSpec((lb, d), lambda p, i: (jnp.where(p == 0, i, last), 0)),
            pl.BlockSpec((d, h1), lambda p, i: (0, 0)),
            pl.BlockSpec((1, h1), lambda p, i: (0, 0)),
            pl.BlockSpec((h1, h2), lambda p, i: (0, 0)),
            pl.BlockSpec((1, h2), lambda p, i: (0, 0)),
        ],
        out_specs=[
            pl.BlockSpec((lb, h2), lambda p, i: (jnp.where(p == 2, i, 0), 0)),
            pl.BlockSpec(memory_space=pltpu.MemorySpace.HBM),
        ],
        out_shape=[
            jax.ShapeDtypeStruct((n, h2), jnp.float32),
            jax.ShapeDtypeStruct((n, n), jnp.bfloat16),
        ],
        scratch_shapes=[
            pltpu.VMEM((n, h1), jnp.bfloat16),
            pltpu.VMEM((n, h2), jnp.bfloat16),
            pltpu.VMEM((2 * lb, n), jnp.bfloat16),
            pltpu.SemaphoreType.DMA((2,)),
            pltpu.SemaphoreType.DMA((2,)),
        ],
        compiler_params=pltpu.CompilerParams(
            dimension_semantics=("arbitrary", "arbitrary"),
            vmem_limit_bytes=63 * 1024 * 1024),
    )(adj, feature, W1, b1r, W2, b2r)

    return x2


# speedup vs baseline: 1.0623x; 1.0623x over previous
import jax
import jax.numpy as jnp
from jax.experimental import pallas as pl
from jax.experimental.pallas import tpu as pltpu


def _pick_row_block(n, pref):
    for cand in pref:
        if n % cand == 0 and (n // cand) % 2 == 1:
            return cand
    return n


def _dinv(deg):
    return jnp.where(deg > 0, jax.lax.rsqrt(deg), 0.0)


def _make_kernel(lb, nlb, align):
    def _kernel(adj_ref, x_ref, w1_ref, b1_ref, w2_ref, b2_ref,
                out_ref, adjb_ref,
                dinv_scr, y1_scr, y2_scr, stg, sem_out, sem_in):
        p = pl.program_id(0)
        i = pl.program_id(1)

        def rows(blk):
            return pl.ds(pl.multiple_of(blk * lb, align), lb)

        def out_copy(blk, buf):
            return pltpu.make_async_copy(
                stg.at[pl.ds(buf * lb, lb), :], adjb_ref.at[rows(blk), :],
                sem_out.at[buf])

        def in_copy(blk, buf):
            return pltpu.make_async_copy(
                adjb_ref.at[rows(blk), :], stg.at[pl.ds(buf * lb, lb), :],
                sem_in.at[buf])

        def p0_body(b):
            @pl.when(i >= 2)
            def _():
                out_copy(i - 2, b).wait()

            stg[pl.ds(b * lb, lb), :] = adj_ref[...].astype(jnp.bfloat16)
            out_copy(i, b).start()
            deg = jnp.sum(stg[pl.ds(b * lb, lb), :], axis=1,
                          dtype=jnp.float32).reshape(lb, 1) + 1.0
            dv = _dinv(deg)
            dinv_scr[rows(i), :] = dv.astype(jnp.bfloat16)
            z = jnp.dot(x_ref[...], w1_ref[...],
                        preferred_element_type=jnp.float32)
            y1_scr[rows(i), :] = (dv * z).astype(jnp.bfloat16)

        @pl.when((p == 0) & (i % 2 == 0))
        def _():
            p0_body(0)

        @pl.when((p == 0) & (i % 2 == 1))
        def _():
            p0_body(1)

        @pl.when((p == 0) & (i == nlb - 1))
        def _():
            out_copy(nlb - 2, (nlb - 2) % 2).wait()
            in_copy(0, 1 - (nlb - 1) % 2).start()

        @pl.when(p > 0)
        def _():
            s = (p - 1) * nlb + i

            @pl.when((p == 1) & (i == 0))
            def _():
                out_copy(nlb - 1, (nlb - 1) % 2).wait()

            def stage_body(cbuf):
                @pl.when(s + 1 < 2 * nlb)
                def _():
                    in_copy((s + 1) % nlb, 1 - cbuf).start()

                in_copy(i, cbuf).wait()
                ab = stg[pl.ds(cbuf * lb, lb), :]
                dinv = dinv_scr[rows(i), :].astype(jnp.float32)

                @pl.when(p == 1)
                def _():
                    acc = jnp.dot(ab, y1_scr[...],
                                  preferred_element_type=jnp.float32)
                    yself = y1_scr[rows(i), :].astype(jnp.float32)
                    x1 = dinv * (acc + yself) + b1_ref[...]
                    x1 = jnp.maximum(x1, 0.0)
                    y2_scr[rows(i), :] = (
                        dinv * jnp.dot(x1, w2_ref[...],
                                       preferred_element_type=jnp.float32)
                    ).astype(jnp.bfloat16)

                @pl.when(p == 2)
                def _():
                    acc = jnp.dot(ab, y2_scr[...],
                                  preferred_element_type=jnp.float32)
                    yself = y2_scr[rows(i), :].astype(jnp.float32)
                    out_ref[...] = dinv * (acc + yself) + b2_ref[...]

            @pl.when((s + 1) % 2 == 0)
            def _():
                stage_body(0)

            @pl.when((s + 1) % 2 == 1)
            def _():
                stage_body(1)

    return _kernel


@jax.jit
def kernel(feature, adj, W1, b1, W2, b2):
    n, d = feature.shape
    h1 = W1.shape[1]
    h2 = W2.shape[1]
    lb = _pick_row_block(n, (400, 200, 80, 40, 16, 8))
    nlb = n // lb
    align = lb & (-lb)

    b1r = b1.reshape(1, h1)
    b2r = b2.reshape(1, h2)

    last = nlb - 1
    x2, _ = pl.pallas_call(
        _make_kernel(lb, nlb, align),
        grid=(3, nlb),
        in_specs=[
            pl.BlockSpec((lb, n), lambda p, i: (jnp.where(p == 0, i, last), 0)),
            pl.BlockSpec((lb, d), lambda p, i: (jnp.where(p == 0, i, last), 0)),
            pl.BlockSpec((d, h1), lambda p, i: (0, 0)),
            pl.BlockSpec((1, h1), lambda p, i: (0, 0)),
            pl.BlockSpec((h1, h2), lambda p, i: (0, 0)),
            pl.BlockSpec((1, h2), lambda p, i: (0, 0)),
        ],
        out_specs=[
            pl.BlockSpec((lb, h2), lambda p, i: (jnp.where(p == 2, i, 0), 0)),
            pl.BlockSpec(memory_space=pltpu.MemorySpace.HBM),
        ],
        out_shape=[
            jax.ShapeDtypeStruct((n, h2), jnp.float32),
            jax.ShapeDtypeStruct((n, n), jnp.bfloat16),
        ],
        scratch_shapes=[
            pltpu.VMEM((n, 1), jnp.bfloat16),
            pltpu.VMEM((n, h1), jnp.bfloat16),
            pltpu.VMEM((n, h2), jnp.bfloat16),
            pltpu.VMEM((2 * lb, n), jnp.bfloat16),
            pltpu.SemaphoreType.DMA((2,)),
            pltpu.SemaphoreType.DMA((2,)),
        ],
        compiler_params=pltpu.CompilerParams(
            dimension_semantics=("arbitrary", "arbitrary"),
            vmem_limit_bytes=63 * 1024 * 1024),
    )(adj, feature, W1, b1r, W2, b2r)

    return x2


# speedup vs baseline: 1.0743x; 1.0114x over previous
import jax
import jax.numpy as jnp
from jax.experimental import pallas as pl
from jax.experimental.pallas import tpu as pltpu


def _pick_row_block(n, pref):
    for cand in pref:
        if n % cand == 0:
            return cand
    return n


def _dinv(deg):
    return jnp.where(deg > 0, jax.lax.rsqrt(deg), 0.0)


def _pass1_kernel(adj_ref, x_ref, w1_ref, deg_ref, adjb_ref, z_ref):
    m = adj_ref.shape[0]
    a = adj_ref[...]
    deg_ref[...] = (jnp.sum(a, axis=1) + 1.0).reshape(m, 1)
    adjb_ref[...] = a.astype(jnp.bfloat16)
    z_ref[...] = jnp.dot(x_ref[...], w1_ref[...], preferred_element_type=jnp.float32)


def _make_layers_kernel(align):
    def _layers_kernel(adj_ref, z_ref, degf_ref, deg_ref, b1_ref, w2_ref, b2_ref,
                       out_ref, y1_scr, y2_scr):
        p = pl.program_id(0)
        i = pl.program_id(1)
        m = adj_ref.shape[0]

        def row_ds():
            return pl.ds(pl.multiple_of(i * m, align), m)

        @pl.when((p == 0) & (i == 0))
        def _():
            y1_scr[...] = (_dinv(degf_ref[...]) * z_ref[...]).astype(jnp.bfloat16)

        @pl.when(p == 0)
        def _():
            dinv = _dinv(deg_ref[...])
            acc = jnp.dot(adj_ref[...], y1_scr[...], preferred_element_type=jnp.float32)
            yself = y1_scr[row_ds(), :].astype(jnp.float32)
            x1 = dinv * (acc + yself) + b1_ref[...]
            x1 = jnp.maximum(x1, 0.0)
            y2_scr[row_ds(), :] = (
                dinv * jnp.dot(x1, w2_ref[...], preferred_element_type=jnp.float32)
            ).astype(jnp.bfloat16)

        @pl.when(p == 1)
        def _():
            dinv = _dinv(deg_ref[...])
            acc = jnp.dot(adj_ref[...], y2_scr[...], preferred_element_type=jnp.float32)
            yself = y2_scr[row_ds(), :].astype(jnp.float32)
            out_ref[...] = dinv * (acc + yself) + b2_ref[...]

    return _layers_kernel


@jax.jit
def kernel(feature, adj, W1, b1, W2, b2):
    n, d = feature.shape
    h1 = W1.shape[1]
    h2 = W2.shape[1]
    mb = _pick_row_block(n, (400, 200, 80, 40, 16, 8))
    nmb = n // mb
    lb = _pick_row_block(n, (400, 200, 80, 40, 16, 8))
    nlb = n // lb
    align = lb & (-lb)

    deg, adjb, z = pl.pallas_call(
        _pass1_kernel,
        grid=(nmb,),
        in_specs=[
            pl.BlockSpec((mb, n), lambda i: (i, 0)),
            pl.BlockSpec((mb, d), lambda i: (i, 0)),
            pl.BlockSpec((d, h1), lambda i: (0, 0)),
        ],
        out_specs=[
            pl.BlockSpec((mb, 1), lambda i: (i, 0)),
            pl.BlockSpec((mb, n), lambda i: (i, 0)),
            pl.BlockSpec((mb, h1), lambda i: (i, 0)),
        ],
        out_shape=[
            jax.ShapeDtypeStruct((n, 1), jnp.float32),
            jax.ShapeDtypeStruct((n, n), jnp.bfloat16),
            jax.ShapeDtypeStruct((n, h1), jnp.float32),
        ],
    )(adj, feature, W1)

    b1r = b1.reshape(1, h1)
    b2r = b2.reshape(1, h2)

    x2 = pl.pallas_call(
        _make_layers_kernel(align),
        grid=(2, nlb),
        in_specs=[
            pl.BlockSpec((lb, n), lambda p, i: (i, 0)),
            pl.BlockSpec((n, h1), lambda p, i: (0, 0)),
            pl.BlockSpec((n, 1), lambda p, i: (0, 0)),
            pl.BlockSpec((lb, 1), lambda p, i: (i, 0)),
            pl.BlockSpec((1, h1), lambda p, i: (0, 0)),
            pl.BlockSpec((h1, h2), lambda p, i: (0, 0)),
            pl.BlockSpec((1, h2), lambda p, i: (0, 0)),
        ],
        out_specs=pl.BlockSpec((lb, h2), lambda p, i: (i, 0)),
        out_shape=jax.ShapeDtypeStruct((n, h2), jnp.float32),
        scratch_shapes=[
            pltpu.VMEM((n, h1), jnp.bfloat16),
            pltpu.VMEM((n, h2), jnp.bfloat16),
        ],
    )(adjb, z, deg, deg, b1r, W2, b2r)

    return x2
